# baseline (speedup 1.0000x reference)
import jax
import jax.numpy as jnp
from jax import lax
from jax.experimental import pallas as pl
from jax.experimental.pallas import tpu as pltpu
from jax.experimental.pallas import tpu_sc as plsc

N_TOKENS = 32768
D_MODEL = 1024
LANES = 16
NUM_WORKERS = 32
TOK_PER_WORKER = N_TOKENS // NUM_WORKERS
T_CHUNK = 16
N_CHUNKS = TOK_PER_WORKER // T_CHUNK
N_SLICES = D_MODEL // LANES
TC_BLOCK = 512


def _bf16_round(v):
    u = lax.bitcast_convert_type(v, jnp.uint32)
    odd = (u >> jnp.uint32(16)) & jnp.uint32(1)
    u = u + (jnp.uint32(0x7FFF) + odd)
    u = u & jnp.uint32(0xFFFF0000)
    return lax.bitcast_convert_type(u, jnp.float32)


def _gate(xb, wg, bgb):
    z = jnp.dot(_bf16_round(xb), wg, preferred_element_type=jnp.float32) + bgb
    s = jax.nn.sigmoid(z)
    w0 = jnp.where(s[:, 0:1] > 0.5, s[:, 0:1], 0.0)
    w1 = jnp.where(s[:, 1:2] > 0.5, s[:, 1:2], 0.0)
    return w0, w1


def _tc_gate_body(x_ref, w_ref, bg_ref, w0_ref):
    w0, _ = _gate(x_ref[...], w_ref[...], bg_ref[...])
    w0_ref[...] = jnp.broadcast_to(w0, (TC_BLOCK, LANES))


def _tc_main_body(x_ref, w_ref, bg_ref, o1_ref, oc_ref):
    xb = x_ref[...]
    w0, w1 = _gate(xb, w_ref[...], bg_ref[...])
    o1_ref[...] = xb * w1
    oc_ref[...] = xb * (w0 + w1)


def _sc_body(x_hbm, wt_hbm, o0_hbm,
             xv0, xv1, wv0, wv1, o0a, o0b, si0, si1, so0, so1):
    wid = lax.axis_index("s") * 2 + lax.axis_index("c")
    base = wid * TOK_PER_WORKER

    def compute_chunk(xv, wv, o0v):
        def token_step(t, carry):
            cw = wv[t, :]
            for i in range(N_SLICES):
                off = i * LANES
                o0v[t, pl.ds(off, LANES)] = xv[t, pl.ds(off, LANES)] * cw
            return carry

        lax.fori_loop(0, T_CHUNK, token_step, 0)

    def half_step(c, xv, wv, o0v, si, so):
        row = base + c * T_CHUNK
        pltpu.make_async_copy(
            x_hbm.at[pl.ds(row, T_CHUNK)], xv, si).wait()
        pltpu.make_async_copy(
            wt_hbm.at[pl.ds(row, T_CHUNK)], wv, si).wait()

        @pl.when(c >= 2)
        def _():
            off = base + (c - 2) * T_CHUNK
            pltpu.make_async_copy(o0v, o0_hbm.at[pl.ds(off, T_CHUNK)], so).wait()

        compute_chunk(xv, wv, o0v)

        @pl.when(c + 2 < N_CHUNKS)
        def _():
            nxt = base + (c + 2) * T_CHUNK
            pltpu.async_copy(x_hbm.at[pl.ds(nxt, T_CHUNK)], xv, si)
            pltpu.async_copy(wt_hbm.at[pl.ds(nxt, T_CHUNK)], wv, si)

        pltpu.async_copy(o0v, o0_hbm.at[pl.ds(row, T_CHUNK)], so)

    pltpu.async_copy(x_hbm.at[pl.ds(base, T_CHUNK)], xv0, si0)
    pltpu.async_copy(wt_hbm.at[pl.ds(base, T_CHUNK)], wv0, si0)
    pltpu.async_copy(x_hbm.at[pl.ds(base + T_CHUNK, T_CHUNK)], xv1, si1)
    pltpu.async_copy(wt_hbm.at[pl.ds(base + T_CHUNK, T_CHUNK)], wv1, si1)

    @pl.loop(0, N_CHUNKS, step=2)
    def _(c):
        half_step(c, xv0, wv0, o0a, si0, so0)
        half_step(c + 1, xv1, wv1, o0b, si1, so1)

    offa = base + (N_CHUNKS - 2) * T_CHUNK
    pltpu.make_async_copy(o0a, o0_hbm.at[pl.ds(offa, T_CHUNK)], so0).wait()
    offb = base + (N_CHUNKS - 1) * T_CHUNK
    pltpu.make_async_copy(o0b, o0_hbm.at[pl.ds(offb, T_CHUNK)], so1).wait()


@jax.jit
def _branch_route(x, wgr, bg2):
    out_sd = jax.ShapeDtypeStruct((N_TOKENS, D_MODEL), jnp.float32)
    grid = (N_TOKENS // TC_BLOCK,)
    x_spec = pl.BlockSpec((TC_BLOCK, D_MODEL), lambda i: (i, 0))
    w_spec = pl.BlockSpec((D_MODEL, 2), lambda i: (0, 0))
    bg_spec = pl.BlockSpec((1, 2), lambda i: (0, 0))

    w0tok = pl.pallas_call(
        _tc_gate_body,
        grid=grid,
        in_specs=[x_spec, w_spec, bg_spec],
        out_specs=pl.BlockSpec((TC_BLOCK, LANES), lambda i: (i, 0)),
        out_shape=jax.ShapeDtypeStruct((N_TOKENS, LANES), jnp.float32),
        compiler_params=pltpu.CompilerParams(
            dimension_semantics=("arbitrary",)),
    )(x, wgr, bg2)

    mesh = plsc.VectorSubcoreMesh(core_axis_name="c", subcore_axis_name="s")
    buf = pltpu.VMEM((T_CHUNK, D_MODEL), jnp.float32)
    wbuf = pltpu.VMEM((T_CHUNK, LANES), jnp.float32)
    x0 = pl.kernel(
        _sc_body,
        mesh=mesh,
        out_type=out_sd,
        compiler_params=pltpu.CompilerParams(needs_layout_passes=False),
        scratch_types=[
            buf, buf,
            wbuf, wbuf,
            buf, buf,
            pltpu.SemaphoreType.DMA,
            pltpu.SemaphoreType.DMA,
            pltpu.SemaphoreType.DMA,
            pltpu.SemaphoreType.DMA,
        ],
    )(x, w0tok)

    x1, comb = pl.pallas_call(
        _tc_main_body,
        grid=grid,
        in_specs=[x_spec, w_spec, bg_spec],
        out_specs=[x_spec, x_spec],
        out_shape=(out_sd, out_sd),
        compiler_params=pltpu.CompilerParams(
            dimension_semantics=("arbitrary",)),
    )(x, wgr, bg2)
    return x0, x1, comb


def kernel(x, Wg, bg):
    wgr = _bf16_round(Wg)
    bg2 = bg.astype(jnp.float32).reshape(1, 2)
    x0, x1, combined = _branch_route(x, wgr, bg2)
    return (x0, x1, combined)


# speedup vs baseline: 1.3269x; 1.3269x over previous
import jax
import jax.numpy as jnp
from jax import lax
from jax.experimental import pallas as pl
from jax.experimental.pallas import tpu as pltpu
from jax.experimental.pallas import tpu_sc as plsc

N_TOKENS = 32768
D_MODEL = 1024
LANES = 16
NUM_WORKERS = 32
TOK_PER_WORKER = N_TOKENS // NUM_WORKERS
T_CHUNK = 16
N_CHUNKS = TOK_PER_WORKER // T_CHUNK
N_SLICES = D_MODEL // LANES
TC_BLOCK = 512


def _bf16_round(v):
    u = lax.bitcast_convert_type(v, jnp.uint32)
    odd = (u >> jnp.uint32(16)) & jnp.uint32(1)
    u = u + (jnp.uint32(0x7FFF) + odd)
    u = u & jnp.uint32(0xFFFF0000)
    return lax.bitcast_convert_type(u, jnp.float32)


def _sc_body(x_hbm, w0_hbm, bg0_hbm, o0_hbm,
             w0v, bg0v, zb, cwb, xv0, xv1, o0a, o0b, si0, si1, so0, so1):
    wid = lax.axis_index("s") * 2 + lax.axis_index("c")
    base = wid * TOK_PER_WORKER

    pltpu.sync_copy(w0_hbm, w0v)
    pltpu.sync_copy(bg0_hbm, bg0v)

    zeros = jnp.zeros((LANES,), jnp.float32)
    lane_iota = lax.iota(jnp.int32, LANES)

    def compute_chunk(xv, o0v):
        def gate_token(t, carry):
            accs = [zeros, zeros, zeros, zeros]
            for i in range(N_SLICES):
                off = i * LANES
                r = _bf16_round(xv[t, pl.ds(off, LANES)])
                accs[i % 4] = accs[i % 4] + r * w0v[pl.ds(off, LANES)]
            zb[pl.ds(t * LANES, LANES)] = (accs[0] + accs[1]) + (accs[2] + accs[3])
            return carry

        lax.fori_loop(0, T_CHUNK, gate_token, 0, unroll=2)

        z = bg0v[...]
        row = lane_iota * LANES
        for j in range(LANES):
            z = z + plsc.load_gather(zb, [row + j])
        s0 = 1.0 / (1.0 + jnp.exp(-z))
        cwb[...] = jnp.where(z > 0.0, s0, 0.0)

        def scale_token(t, carry):
            cw = plsc.load_gather(cwb, [jnp.full((LANES,), t, jnp.int32)])
            for i in range(N_SLICES):
                off = i * LANES
                o0v[t, pl.ds(off, LANES)] = xv[t, pl.ds(off, LANES)] * cw
            return carry

        lax.fori_loop(0, T_CHUNK, scale_token, 0, unroll=2)

    def half_step(c, xv, o0v, si, so):
        row = base + c * T_CHUNK
        pltpu.make_async_copy(x_hbm.at[pl.ds(row, T_CHUNK)], xv, si).wait()

        @pl.when(c >= 2)
        def _():
            off = base + (c - 2) * T_CHUNK
            pltpu.make_async_copy(o0v, o0_hbm.at[pl.ds(off, T_CHUNK)], so).wait()

        compute_chunk(xv, o0v)

        @pl.when(c + 2 < N_CHUNKS)
        def _():
            nxt = base + (c + 2) * T_CHUNK
            pltpu.async_copy(x_hbm.at[pl.ds(nxt, T_CHUNK)], xv, si)

        pltpu.async_copy(o0v, o0_hbm.at[pl.ds(row, T_CHUNK)], so)

    pltpu.async_copy(x_hbm.at[pl.ds(base, T_CHUNK)], xv0, si0)
    pltpu.async_copy(x_hbm.at[pl.ds(base + T_CHUNK, T_CHUNK)], xv1, si1)

    @pl.loop(0, N_CHUNKS, step=2)
    def _(c):
        half_step(c, xv0, o0a, si0, so0)
        half_step(c + 1, xv1, o0b, si1, so1)

    offa = base + (N_CHUNKS - 2) * T_CHUNK
    pltpu.make_async_copy(o0a, o0_hbm.at[pl.ds(offa, T_CHUNK)], so0).wait()
    offb = base + (N_CHUNKS - 1) * T_CHUNK
    pltpu.make_async_copy(o0b, o0_hbm.at[pl.ds(offb, T_CHUNK)], so1).wait()


def _tc_body(x_ref, w_ref, bg_ref, o1_ref, oc_ref):
    xb = x_ref[...]
    z = jnp.dot(_bf16_round(xb), w_ref[...],
                preferred_element_type=jnp.float32) + bg_ref[...]
    s = jax.nn.sigmoid(z)
    w0 = jnp.where(s[:, 0:1] > 0.5, s[:, 0:1], 0.0)
    w1 = jnp.where(s[:, 1:2] > 0.5, s[:, 1:2], 0.0)
    o1_ref[...] = xb * w1
    oc_ref[...] = xb * (w0 + w1)


@jax.jit
def _branch_route(x, w0, bg0, wgr, bg2):
    out_sd = jax.ShapeDtypeStruct((N_TOKENS, D_MODEL), jnp.float32)
    mesh = plsc.VectorSubcoreMesh(core_axis_name="c", subcore_axis_name="s")
    buf = pltpu.VMEM((T_CHUNK, D_MODEL), jnp.float32)
    x0 = pl.kernel(
        _sc_body,
        mesh=mesh,
        out_type=out_sd,
        compiler_params=pltpu.CompilerParams(needs_layout_passes=False),
        scratch_types=[
            pltpu.VMEM((D_MODEL,), jnp.float32),
            pltpu.VMEM((LANES,), jnp.float32),
            pltpu.VMEM((T_CHUNK * LANES,), jnp.float32),
            pltpu.VMEM((LANES,), jnp.float32),
            buf, buf,
            buf, buf,
            pltpu.SemaphoreType.DMA,
            pltpu.SemaphoreType.DMA,
            pltpu.SemaphoreType.DMA,
            pltpu.SemaphoreType.DMA,
        ],
    )(x, w0, bg0)

    grid = (N_TOKENS // TC_BLOCK,)
    x_spec = pl.BlockSpec((TC_BLOCK, D_MODEL), lambda i: (i, 0))
    x1, comb = pl.pallas_call(
        _tc_body,
        grid=grid,
        in_specs=[
            x_spec,
            pl.BlockSpec((D_MODEL, 2), lambda i: (0, 0)),
            pl.BlockSpec((1, 2), lambda i: (0, 0)),
        ],
        out_specs=[x_spec, x_spec],
        out_shape=(out_sd, out_sd),
        compiler_params=pltpu.CompilerParams(
            dimension_semantics=("arbitrary",)),
    )(x, wgr, bg2)
    return x0, x1, comb


def kernel(x, Wg, bg):
    wgr = _bf16_round(Wg)
    w0 = wgr[:, 0]
    bg0 = jnp.full((LANES,), bg[0], jnp.float32)
    bg2 = bg.astype(jnp.float32).reshape(1, 2)
    x0, x1, combined = _branch_route(x, w0, bg0, wgr, bg2)
    return (x0, x1, combined)


# speedup vs baseline: 1.3302x; 1.0025x over previous
import jax
import jax.numpy as jnp
from jax import lax
from jax.experimental import pallas as pl
from jax.experimental.pallas import tpu as pltpu
from jax.experimental.pallas import tpu_sc as plsc

N_TOKENS = 32768
D_MODEL = 1024
LANES = 16
NUM_WORKERS = 32
TOK_PER_WORKER = N_TOKENS // NUM_WORKERS
T_CHUNK = 8
N_CHUNKS = TOK_PER_WORKER // T_CHUNK
N_SLICES = D_MODEL // LANES
N_BUF = 4
TC_BLOCK = 512


def _bf16_round(v):
    u = lax.bitcast_convert_type(v, jnp.uint32)
    odd = (u >> jnp.uint32(16)) & jnp.uint32(1)
    u = u + (jnp.uint32(0x7FFF) + odd)
    u = u & jnp.uint32(0xFFFF0000)
    return lax.bitcast_convert_type(u, jnp.float32)


def _sc_body(x_hbm, w0_hbm, bg0_hbm, o0_hbm,
             w0v, bg0v,
             xv0, xv1, xv2, xv3, o0a, o0b, o0c, o0d,
             si0, si1, si2, si3, so0, so1, so2, so3):
    wid = lax.axis_index("s") * 2 + lax.axis_index("c")
    base = wid * TOK_PER_WORKER

    pltpu.sync_copy(w0_hbm, w0v)
    pltpu.sync_copy(bg0_hbm, bg0v)

    zeros = jnp.zeros((LANES,), jnp.float32)
    xvs = [xv0, xv1, xv2, xv3]
    ovs = [o0a, o0b, o0c, o0d]
    sis = [si0, si1, si2, si3]
    sos = [so0, so1, so2, so3]

    def compute_chunk(xv, o0v):
        def token_step(t, carry):
            accs = [zeros, zeros, zeros, zeros]
            for i in range(N_SLICES):
                off = i * LANES
                r = _bf16_round(xv[t, pl.ds(off, LANES)])
                accs[i % 4] = accs[i % 4] + r * w0v[pl.ds(off, LANES)]
            b = (accs[0] + accs[1]) + (accs[2] + accs[3])
            z0 = jnp.full((LANES,), jnp.sum(b), jnp.float32) + bg0v[...]
            s0 = 1.0 / (1.0 + jnp.exp(-z0))
            c0 = jnp.where(z0 > 0.0, s0, 0.0)
            for i in range(N_SLICES):
                off = i * LANES
                o0v[t, pl.ds(off, LANES)] = xv[t, pl.ds(off, LANES)] * c0
            return carry

        lax.fori_loop(0, T_CHUNK, token_step, 0, unroll=2)

    def half_step(c, xv, o0v, si, so):
        row = base + c * T_CHUNK
        pltpu.make_async_copy(x_hbm.at[pl.ds(row, T_CHUNK)], xv, si).wait()

        @pl.when(c >= N_BUF)
        def _():
            off = base + (c - N_BUF) * T_CHUNK
            pltpu.make_async_copy(o0v, o0_hbm.at[pl.ds(off, T_CHUNK)], so).wait()

        compute_chunk(xv, o0v)

        @pl.when(c + N_BUF < N_CHUNKS)
        def _():
            nxt = base + (c + N_BUF) * T_CHUNK
            pltpu.async_copy(x_hbm.at[pl.ds(nxt, T_CHUNK)], xv, si)

        pltpu.async_copy(o0v, o0_hbm.at[pl.ds(row, T_CHUNK)], so)

    for b in range(N_BUF):
        pltpu.async_copy(
            x_hbm.at[pl.ds(base + b * T_CHUNK, T_CHUNK)], xvs[b], sis[b])

    @pl.loop(0, N_CHUNKS, step=N_BUF)
    def _(c):
        for b in range(N_BUF):
            half_step(c + b, xvs[b], ovs[b], sis[b], sos[b])

    for b in range(N_BUF):
        off = base + (N_CHUNKS - N_BUF + b) * T_CHUNK
        pltpu.make_async_copy(
            ovs[b], o0_hbm.at[pl.ds(off, T_CHUNK)], sos[b]).wait()


def _tc_body(x_ref, w_ref, bg_ref, o1_ref, oc_ref):
    xb = x_ref[...]
    z = jnp.dot(_bf16_round(xb), w_ref[...],
                preferred_element_type=jnp.float32) + bg_ref[...]
    s = jax.nn.sigmoid(z)
    w0 = jnp.where(s[:, 0:1] > 0.5, s[:, 0:1], 0.0)
    w1 = jnp.where(s[:, 1:2] > 0.5, s[:, 1:2], 0.0)
    o1_ref[...] = xb * w1
    oc_ref[...] = xb * (w0 + w1)


@jax.jit
def _branch_route(x, w0, bg0, wgr, bg2):
    out_sd = jax.ShapeDtypeStruct((N_TOKENS, D_MODEL), jnp.float32)
    mesh = plsc.VectorSubcoreMesh(core_axis_name="c", subcore_axis_name="s")
    buf = pltpu.VMEM((T_CHUNK, D_MODEL), jnp.float32)
    x0 = pl.kernel(
        _sc_body,
        mesh=mesh,
        out_type=out_sd,
        compiler_params=pltpu.CompilerParams(needs_layout_passes=False),
        scratch_types=(
            [pltpu.VMEM((D_MODEL,), jnp.float32),
             pltpu.VMEM((LANES,), jnp.float32)]
            + [buf] * 8
            + [pltpu.SemaphoreType.DMA] * 8
        ),
    )(x, w0, bg0)

    grid = (N_TOKENS // TC_BLOCK,)
    x_spec = pl.BlockSpec((TC_BLOCK, D_MODEL), lambda i: (i, 0))
    x1, comb = pl.pallas_call(
        _tc_body,
        grid=grid,
        in_specs=[
            x_spec,
            pl.BlockSpec((D_MODEL, 2), lambda i: (0, 0)),
            pl.BlockSpec((1, 2), lambda i: (0, 0)),
        ],
        out_specs=[x_spec, x_spec],
        out_shape=(out_sd, out_sd),
        compiler_params=pltpu.CompilerParams(
            dimension_semantics=("arbitrary",)),
    )(x, wgr, bg2)
    return x0, x1, comb


def kernel(x, Wg, bg):
    wgr = _bf16_round(Wg)
    w0 = wgr[:, 0]
    bg0 = jnp.full((LANES,), bg[0], jnp.float32)
    bg2 = bg.astype(jnp.float32).reshape(1, 2)
    x0, x1, combined = _branch_route(x, w0, bg0, wgr, bg2)
    return (x0, x1, combined)


# speedup vs baseline: 1.3306x; 1.0003x over previous
import jax
import jax.numpy as jnp
from jax import lax
from jax.experimental import pallas as pl
from jax.experimental.pallas import tpu as pltpu
from jax.experimental.pallas import tpu_sc as plsc

N_TOKENS = 32768
D_MODEL = 1024
LANES = 16
NUM_WORKERS = 32
TOK_PER_WORKER = N_TOKENS // NUM_WORKERS
T_CHUNK = 8
N_CHUNKS = TOK_PER_WORKER // T_CHUNK
N_SLICES = D_MODEL // LANES
N_BUF = 4
TC_BLOCK = 512


def _bf16_round(v):
    u = lax.bitcast_convert_type(v, jnp.uint32)
    odd = (u >> jnp.uint32(16)) & jnp.uint32(1)
    u = u + (jnp.uint32(0x7FFF) + odd)
    u = u & jnp.uint32(0xFFFF0000)
    return lax.bitcast_convert_type(u, jnp.float32)


def _sc_body(x_hbm, w0_hbm, bg0_hbm, o0_hbm,
             w0v, bg0v,
             xv0, xv1, xv2, xv3, o0a, o0b, o0c, o0d,
             si0, si1, si2, si3, so0, so1, so2, so3):
    wid = lax.axis_index("s") * 2 + lax.axis_index("c")
    base = wid * TOK_PER_WORKER

    pltpu.sync_copy(w0_hbm, w0v)
    pltpu.sync_copy(bg0_hbm, bg0v)

    zeros = jnp.zeros((LANES,), jnp.float32)
    xvs = [xv0, xv1, xv2, xv3]
    ovs = [o0a, o0b, o0c, o0d]
    sis = [si0, si1, si2, si3]
    sos = [so0, so1, so2, so3]

    def compute_chunk(xv, o0v):
        def token_step(t, carry):
            accs = [zeros, zeros, zeros, zeros]
            for i in range(N_SLICES):
                off = i * LANES
                r = _bf16_round(xv[t, pl.ds(off, LANES)])
                accs[i % 4] = accs[i % 4] + r * w0v[pl.ds(off, LANES)]
            b = (accs[0] + accs[1]) + (accs[2] + accs[3])
            z0 = jnp.full((LANES,), jnp.sum(b), jnp.float32) + bg0v[...]
            s0 = 1.0 / (1.0 + jnp.exp(-z0))
            c0 = jnp.where(z0 > 0.0, s0, 0.0)
            for i in range(N_SLICES):
                off = i * LANES
                o0v[t, pl.ds(off, LANES)] = xv[t, pl.ds(off, LANES)] * c0
            return carry

        lax.fori_loop(0, T_CHUNK, token_step, 0, unroll=2)

    def half_step(c, xv, o0v, si, so):
        row = base + c * T_CHUNK
        pltpu.make_async_copy(x_hbm.at[pl.ds(row, T_CHUNK)], xv, si).wait()

        @pl.when(c >= N_BUF)
        def _():
            off = base + (c - N_BUF) * T_CHUNK
            pltpu.make_async_copy(o0v, o0_hbm.at[pl.ds(off, T_CHUNK)], so).wait()

        compute_chunk(xv, o0v)

        @pl.when(c + N_BUF < N_CHUNKS)
        def _():
            nxt = base + (c + N_BUF) * T_CHUNK
            pltpu.async_copy(x_hbm.at[pl.ds(nxt, T_CHUNK)], xv, si)

        pltpu.async_copy(o0v, o0_hbm.at[pl.ds(row, T_CHUNK)], so)

    for b in range(N_BUF):
        pltpu.async_copy(
            x_hbm.at[pl.ds(base + b * T_CHUNK, T_CHUNK)], xvs[b], sis[b])

    @pl.loop(0, N_CHUNKS, step=N_BUF)
    def _(c):
        for b in range(N_BUF):
            half_step(c + b, xvs[b], ovs[b], sis[b], sos[b])

    for b in range(N_BUF):
        off = base + (N_CHUNKS - N_BUF + b) * T_CHUNK
        pltpu.make_async_copy(
            ovs[b], o0_hbm.at[pl.ds(off, T_CHUNK)], sos[b]).wait()


def _tc_body(x_ref, w_ref, bg_ref, o1_ref, oc_ref):
    xb = x_ref[...]
    z = jnp.dot(_bf16_round(xb), w_ref[...],
                preferred_element_type=jnp.float32) + bg_ref[...]
    s = jax.nn.sigmoid(z)
    w0 = jnp.where(s[:, 0:1] > 0.5, s[:, 0:1], 0.0)
    w1 = jnp.where(s[:, 1:2] > 0.5, s[:, 1:2], 0.0)
    o1_ref[...] = xb * w1
    oc_ref[...] = xb * (w0 + w1)


@jax.jit
def _branch_route(x, w0, bg0, wgr, bg2):
    out_sd = jax.ShapeDtypeStruct((N_TOKENS, D_MODEL), jnp.float32)

    grid = (N_TOKENS // TC_BLOCK,)
    x_spec = pl.BlockSpec((TC_BLOCK, D_MODEL), lambda i: (i, 0))
    x1, comb = pl.pallas_call(
        _tc_body,
        grid=grid,
        in_specs=[
            x_spec,
            pl.BlockSpec((D_MODEL, 2), lambda i: (0, 0)),
            pl.BlockSpec((1, 2), lambda i: (0, 0)),
        ],
        out_specs=[x_spec, x_spec],
        out_shape=(out_sd, out_sd),
        compiler_params=pltpu.CompilerParams(
            dimension_semantics=("arbitrary",)),
    )(x, wgr, bg2)

    mesh = plsc.VectorSubcoreMesh(core_axis_name="c", subcore_axis_name="s")
    buf = pltpu.VMEM((T_CHUNK, D_MODEL), jnp.float32)
    x0 = pl.kernel(
        _sc_body,
        mesh=mesh,
        out_type=out_sd,
        compiler_params=pltpu.CompilerParams(needs_layout_passes=False),
        scratch_types=(
            [pltpu.VMEM((D_MODEL,), jnp.float32),
             pltpu.VMEM((LANES,), jnp.float32)]
            + [buf] * 8
            + [pltpu.SemaphoreType.DMA] * 8
        ),
    )(x, w0, bg0)
    return x0, x1, comb


def kernel(x, Wg, bg):
    wgr = _bf16_round(Wg)
    w0 = wgr[:, 0]
    bg0 = jnp.full((LANES,), bg[0], jnp.float32)
    bg2 = bg.astype(jnp.float32).reshape(1, 2)
    x0, x1, combined = _branch_route(x, w0, bg0, wgr, bg2)
    return (x0, x1, combined)


# speedup vs baseline: 1.3361x; 1.0042x over previous
import jax
import jax.numpy as jnp
from jax import lax
from jax.experimental import pallas as pl
from jax.experimental.pallas import tpu as pltpu
from jax.experimental.pallas import tpu_sc as plsc

N_TOKENS = 32768
D_MODEL = 1024
LANES = 16
NUM_WORKERS = 32
TOK_PER_WORKER = N_TOKENS // NUM_WORKERS
T_CHUNK = 16
N_CHUNKS = TOK_PER_WORKER // T_CHUNK
N_SLICES = D_MODEL // LANES
UNROLL = 8
TC_BLOCK = 512


def _bf16_round(v):
    u = lax.bitcast_convert_type(v, jnp.uint32)
    odd = (u >> jnp.uint32(16)) & jnp.uint32(1)
    u = u + (jnp.uint32(0x7FFF) + odd)
    u = u & jnp.uint32(0xFFFF0000)
    return lax.bitcast_convert_type(u, jnp.float32)


def _sc_body(x_hbm, w0_hbm, bg0_hbm, o0_hbm,
             w0v, bg0v, xv0, xv1, o0a, o0b, si0, si1, so0, so1):
    wid = lax.axis_index("s") * 2 + lax.axis_index("c")
    base = wid * TOK_PER_WORKER

    pltpu.sync_copy(w0_hbm, w0v)
    pltpu.sync_copy(bg0_hbm, bg0v)

    zeros = jnp.zeros((LANES,), jnp.float32)

    def compute_chunk(xv, o0v):
        def token_step(t, carry):
            accs = [zeros, zeros, zeros, zeros]
            for i in range(N_SLICES):
                off = i * LANES
                r = _bf16_round(xv[t, pl.ds(off, LANES)])
                accs[i % 4] = accs[i % 4] + r * w0v[pl.ds(off, LANES)]
            b = (accs[0] + accs[1]) + (accs[2] + accs[3])
            z0 = jnp.full((LANES,), jnp.sum(b), jnp.float32) + bg0v[...]
            s0 = 1.0 / (1.0 + jnp.exp(-z0))
            c0 = jnp.where(z0 > 0.0, s0, 0.0)

            for i in range(N_SLICES):
                off = i * LANES
                o0v[t, pl.ds(off, LANES)] = xv[t, pl.ds(off, LANES)] * c0
            return carry

        lax.fori_loop(0, T_CHUNK, token_step, 0)

    def half_step(c, xv, o0v, si, so):
        pltpu.make_async_copy(
            x_hbm.at[pl.ds(base + c * T_CHUNK, T_CHUNK)], xv, si).wait()

        @pl.when(c >= 2)
        def _():
            off = base + (c - 2) * T_CHUNK
            pltpu.make_async_copy(o0v, o0_hbm.at[pl.ds(off, T_CHUNK)], so).wait()

        compute_chunk(xv, o0v)

        @pl.when(c + 2 < N_CHUNKS)
        def _():
            pltpu.async_copy(
                x_hbm.at[pl.ds(base + (c + 2) * T_CHUNK, T_CHUNK)], xv, si)

        pltpu.async_copy(o0v, o0_hbm.at[pl.ds(base + c * T_CHUNK, T_CHUNK)], so)

    pltpu.async_copy(x_hbm.at[pl.ds(base, T_CHUNK)], xv0, si0)
    pltpu.async_copy(x_hbm.at[pl.ds(base + T_CHUNK, T_CHUNK)], xv1, si1)

    @pl.loop(0, N_CHUNKS, step=2)
    def _(c):
        half_step(c, xv0, o0a, si0, so0)
        half_step(c + 1, xv1, o0b, si1, so1)

    offa = base + (N_CHUNKS - 2) * T_CHUNK
    pltpu.make_async_copy(o0a, o0_hbm.at[pl.ds(offa, T_CHUNK)], so0).wait()
    offb = base + (N_CHUNKS - 1) * T_CHUNK
    pltpu.make_async_copy(o0b, o0_hbm.at[pl.ds(offb, T_CHUNK)], so1).wait()


def _tc_body(x_ref, w_ref, bg_ref, o1_ref, oc_ref):
    xb = x_ref[...]
    xr = _bf16_round(xb)
    wg = w_ref[...]
    z = jnp.dot(xr, wg, preferred_element_type=jnp.float32) + bg_ref[...]
    s = jax.nn.sigmoid(z)
    w0 = jnp.where(s[:, 0:1] > 0.5, s[:, 0:1], 0.0)
    w1 = jnp.where(s[:, 1:2] > 0.5, s[:, 1:2], 0.0)
    o1_ref[...] = xb * w1
    oc_ref[...] = xb * (w0 + w1)


@jax.jit
def _branch_route(x, w0, bg0, wgr, bg2):
    out_sd = jax.ShapeDtypeStruct((N_TOKENS, D_MODEL), jnp.float32)
    mesh = plsc.VectorSubcoreMesh(core_axis_name="c", subcore_axis_name="s")
    buf = pltpu.VMEM((T_CHUNK, D_MODEL), jnp.float32)
    x0 = pl.kernel(
        _sc_body,
        mesh=mesh,
        out_type=out_sd,
        compiler_params=pltpu.CompilerParams(needs_layout_passes=False),
        scratch_types=[
            pltpu.VMEM((D_MODEL,), jnp.float32),
            pltpu.VMEM((LANES,), jnp.float32),
            buf, buf,
            buf, buf,
            pltpu.SemaphoreType.DMA,
            pltpu.SemaphoreType.DMA,
            pltpu.SemaphoreType.DMA,
            pltpu.SemaphoreType.DMA,
        ],
    )(x, w0, bg0)

    grid = (N_TOKENS // TC_BLOCK,)
    x1, comb = pl.pallas_call(
        _tc_body,
        grid=grid,
        in_specs=[
            pl.BlockSpec((TC_BLOCK, D_MODEL), lambda i: (i, 0)),
            pl.BlockSpec((D_MODEL, 2), lambda i: (0, 0)),
            pl.BlockSpec((1, 2), lambda i: (0, 0)),
        ],
        out_specs=[
            pl.BlockSpec((TC_BLOCK, D_MODEL), lambda i: (i, 0)),
            pl.BlockSpec((TC_BLOCK, D_MODEL), lambda i: (i, 0)),
        ],
        out_shape=(out_sd, out_sd),
        compiler_params=pltpu.CompilerParams(
            dimension_semantics=("arbitrary",)),
    )(x, wgr, bg2)
    return x0, x1, comb


def kernel(x, Wg, bg):
    wgr = _bf16_round(Wg)
    w0 = wgr[:, 0]
    bg0 = jnp.full((LANES,), bg[0], jnp.float32)
    bg2 = bg.astype(jnp.float32).reshape(1, 2)
    x0, x1, combined = _branch_route(x, w0, bg0, wgr, bg2)
    return (x0, x1, combined)
